# baseline (speedup 1.0000x reference)
import functools

import jax
import jax.numpy as jnp
from jax import lax
from jax.experimental import pallas as pl
from jax.experimental.pallas import tpu as pltpu
from jax.experimental.pallas import tpu_sc as plsc

N = 16384
D = 4096
K = 512
NC = 2
NS = 16
L = 16
CHUNK = N // NS
BR = 16
NBLK = K // BR

_ONE = lambda: jnp.full((L,), 1, jnp.int32)
_ZERO = lambda: jnp.full((L,), 0, jnp.int32)


def _idx_body(labels_hbm, nm_hbm, idx_hbm,
              lab_v, nm_v, vals_flat, slots_v, cnts_all_v,
              counts_sh, idx_sh, sem_s):
    cid = lax.axis_index("c")
    sid = lax.axis_index("s")

    pltpu.sync_copy(labels_hbm.at[pl.ds(sid * CHUNK, CHUNK)], lab_v)
    pltpu.sync_copy(nm_hbm, nm_v)

    def _count(r, acc):
        for t in range(8):
            j = r * 8 + t
            v = lab_v[pl.ds(j * L, L)]
            vals_flat[pl.ds(j * L, L)] = (
                sid * CHUNK + j * L + lax.iota(jnp.int32, L))
            acc = acc + jnp.where(v != -100, _ONE(), _ZERO())
        return acc

    acc0 = lax.fori_loop(0, 8, _count, jnp.zeros((L,), jnp.int32))
    count = jnp.sum(acc0)
    nm = nm_v[...][0]
    nm_eff = jnp.minimum(jnp.maximum(nm, K), N)

    nm_v[...] = jnp.full((L,), count, jnp.int32)
    pltpu.sync_copy(nm_v, counts_sh.at[sid])
    plsc.subcore_barrier()
    pltpu.sync_copy(counts_sh, cnts_all_v)

    cvals = [cnts_all_v[j][0] for j in range(NS)]
    m = functools.reduce(lambda a, b: a + b, cvals)
    zero = jnp.int32(0)
    oa = functools.reduce(
        lambda a, b: a + b,
        [jnp.where(jnp.int32(j) > sid, cvals[j], zero) for j in range(NS)])
    cs = functools.reduce(
        lambda a, b: a + b,
        [jnp.where(jnp.int32(j) == sid, cvals[j], zero) for j in range(NS)])
    ob = m - oa - cs
    za = (N - (sid + 1) * CHUNK) - oa

    fast = jnp.logical_and(
        jnp.logical_and(oa == 0, za == 0),
        jnp.logical_and(cs == CHUNK, nm_eff == K))

    @pl.when(fast)
    def _linear_indices():
        pltpu.sync_copy(vals_flat.at[pl.ds(CHUNK - K, K)],
                        idx_sh.at[pl.ds(0, K)])

    @pl.when(jnp.logical_and(jnp.minimum(oa, m + za) < nm_eff,
                             jnp.logical_not(fast)))
    def _scatter_indices():
        def _fill(j, carry):
            v = lab_v[pl.ds(j * L, L)]
            is1 = v != -100
            ones = jnp.where(is1, _ONE(), _ZERO())
            up_incl = carry + plsc.cumsum(ones)
            ones_after = m - up_incl
            ivec = vals_flat[pl.ds(j * L, L)]
            zeros_after = (N - 1 - ivec) - ones_after
            rank = jnp.where(is1, ones_after, m + zeros_after)
            slot = (nm_eff - 1) - rank
            dump = K + lax.iota(jnp.int32, L)
            scat = jnp.where(slot >= 0, jnp.where(slot < K, slot, dump),
                             dump)
            slots_v[j // 8, pl.ds((j % 8) * L, L)] = scat
            return carry + jnp.sum(ones)

        lax.fori_loop(0, CHUNK // L, _fill, ob)

        def _scat(r, carry):
            pltpu.async_copy(vals_flat.at[pl.ds(r * 128, 128)],
                             idx_sh.at[slots_v.at[r]], sem_s).wait()
            return carry

        lax.fori_loop(0, 8, _scat, jnp.int32(0))

    plsc.subcore_barrier()

    @pl.when(jnp.logical_and(cid == 0, sid == 0))
    def _emit():
        pltpu.sync_copy(idx_sh.at[pl.ds(0, K)], idx_hbm)


_sc_idx = pl.kernel(
    _idx_body,
    out_type=jax.ShapeDtypeStruct((K,), jnp.int32),
    mesh=plsc.VectorSubcoreMesh(core_axis_name="c", subcore_axis_name="s"),
    compiler_params=pltpu.CompilerParams(needs_layout_passes=False),
    scratch_types=[
        pltpu.VMEM((CHUNK,), jnp.int32),
        pltpu.VMEM((L,), jnp.int32),
        pltpu.VMEM((CHUNK,), jnp.int32),
        pltpu.VMEM((8, 128), jnp.int32),
        pltpu.VMEM((NS, L), jnp.int32),
        pltpu.VMEM_SHARED((NS, L), jnp.int32),
        pltpu.VMEM_SHARED((K + L,), jnp.int32),
        pltpu.SemaphoreType.DMA,
    ],
)


def _tc_body(idx_sm, out_hbm, loss_ref, sel_ref, bufa, bufb, sema, semb):
    i = pl.program_id(0)

    def _issue(block, buf, sem):
        for r in range(BR):
            pltpu.make_async_copy(
                out_hbm.at[pl.ds(idx_sm[block * BR + r], 1), :],
                buf.at[pl.ds(r, 1), :], sem).start()

    def _drain(block, buf, sem):
        for r in range(BR):
            pltpu.make_async_copy(
                out_hbm.at[pl.ds(idx_sm[block * BR + r], 1), :],
                buf.at[pl.ds(r, 1), :], sem).wait()

    @pl.when(i == 0)
    def _prologue():
        loss_ref[0, 0] = jnp.float32(0.0)
        _issue(0, bufa, sema)

    even = i % 2 == 0

    @pl.when(jnp.logical_and(i + 1 < NBLK, even))
    def _next_b():
        _issue(i + 1, bufb, semb)

    @pl.when(jnp.logical_and(i + 1 < NBLK, jnp.logical_not(even)))
    def _next_a():
        _issue(i + 1, bufa, sema)

    @pl.when(even)
    def _consume_a():
        _drain(i, bufa, sema)
        sel_ref[...] = bufa[...]
        loss_ref[0, 0] += jnp.sum(bufa[...])

    @pl.when(jnp.logical_not(even))
    def _consume_b():
        _drain(i, bufb, semb)
        sel_ref[...] = bufb[...]
        loss_ref[0, 0] += jnp.sum(bufb[...])


_tc_gather = pl.pallas_call(
    _tc_body,
    grid_spec=pltpu.PrefetchScalarGridSpec(
        num_scalar_prefetch=1,
        grid=(NBLK,),
        in_specs=[pl.BlockSpec(memory_space=pltpu.MemorySpace.HBM)],
        out_specs=[
            pl.BlockSpec((1, 1), lambda i, idx: (0, 0),
                         memory_space=pltpu.MemorySpace.SMEM),
            pl.BlockSpec((BR, D), lambda i, idx: (i, 0)),
        ],
        scratch_shapes=[
            pltpu.VMEM((BR, D), jnp.float32),
            pltpu.VMEM((BR, D), jnp.float32),
            pltpu.SemaphoreType.DMA,
            pltpu.SemaphoreType.DMA,
        ],
    ),
    out_shape=(
        jax.ShapeDtypeStruct((1, 1), jnp.float32),
        jax.ShapeDtypeStruct((K, D), jnp.float32),
    ),
)


def kernel(outputs, labels, num_masks):
    nm_arr = jnp.full((L,), num_masks, dtype=jnp.int32)
    idx = _sc_idx(labels, nm_arr)
    loss_sum, sel = _tc_gather(idx, outputs)
    loss = loss_sum[0, 0] * jnp.float32(1.0 / (K * D))
    return loss, sel


# speedup vs baseline: 1.1374x; 1.1374x over previous
import functools

import jax
import jax.numpy as jnp
from jax import lax
from jax.experimental import pallas as pl
from jax.experimental.pallas import tpu as pltpu
from jax.experimental.pallas import tpu_sc as plsc

N = 16384
D = 4096
K = 512
NC = 2
NS = 16
L = 16
CHUNK = N // NS
BR = 16
NBLK = K // BR

_ONE = lambda: jnp.full((L,), 1, jnp.int32)
_ZERO = lambda: jnp.full((L,), 0, jnp.int32)


def _idx_body(labels_hbm, nm_hbm, idx_hbm,
              lab_v, nm_v, vals_flat, slots_v, cnts_all_v,
              counts_sh, idx_sh, sem_s):
    cid = lax.axis_index("c")
    sid = lax.axis_index("s")

    pltpu.sync_copy(labels_hbm.at[pl.ds(sid * CHUNK, CHUNK)], lab_v)
    pltpu.sync_copy(nm_hbm, nm_v)

    def _count(r, acc):
        for t in range(8):
            j = r * 8 + t
            v = lab_v[pl.ds(j * L, L)]
            vals_flat[pl.ds(j * L, L)] = (
                sid * CHUNK + j * L + lax.iota(jnp.int32, L))
            acc = acc + jnp.where(v != -100, _ONE(), _ZERO())
        return acc

    acc0 = lax.fori_loop(0, 8, _count, jnp.zeros((L,), jnp.int32))
    count = jnp.sum(acc0)
    nm = nm_v[...][0]
    nm_eff = jnp.minimum(jnp.maximum(nm, K), N)

    nm_v[...] = jnp.full((L,), count, jnp.int32)
    pltpu.sync_copy(nm_v, counts_sh.at[sid])
    plsc.subcore_barrier()
    pltpu.sync_copy(counts_sh, cnts_all_v)

    cvals = [cnts_all_v[j][0] for j in range(NS)]
    m = functools.reduce(lambda a, b: a + b, cvals)
    zero = jnp.int32(0)
    oa = functools.reduce(
        lambda a, b: a + b,
        [jnp.where(jnp.int32(j) > sid, cvals[j], zero) for j in range(NS)])
    cs = functools.reduce(
        lambda a, b: a + b,
        [jnp.where(jnp.int32(j) == sid, cvals[j], zero) for j in range(NS)])
    ob = m - oa - cs
    za = (N - (sid + 1) * CHUNK) - oa

    fast = jnp.logical_and(
        jnp.logical_and(oa == 0, za == 0),
        jnp.logical_and(cs == CHUNK, nm_eff == K))

    @pl.when(fast)
    def _linear_indices():
        pltpu.sync_copy(vals_flat.at[pl.ds(CHUNK - K, K)],
                        idx_sh.at[pl.ds(0, K)])

    @pl.when(jnp.logical_and(jnp.minimum(oa, m + za) < nm_eff,
                             jnp.logical_not(fast)))
    def _scatter_indices():
        def _fill(j, carry):
            v = lab_v[pl.ds(j * L, L)]
            is1 = v != -100
            ones = jnp.where(is1, _ONE(), _ZERO())
            up_incl = carry + plsc.cumsum(ones)
            ones_after = m - up_incl
            ivec = vals_flat[pl.ds(j * L, L)]
            zeros_after = (N - 1 - ivec) - ones_after
            rank = jnp.where(is1, ones_after, m + zeros_after)
            slot = (nm_eff - 1) - rank
            dump = K + lax.iota(jnp.int32, L)
            scat = jnp.where(slot >= 0, jnp.where(slot < K, slot, dump),
                             dump)
            slots_v[j // 8, pl.ds((j % 8) * L, L)] = scat
            return carry + jnp.sum(ones)

        lax.fori_loop(0, CHUNK // L, _fill, ob)

        def _scat(r, carry):
            pltpu.async_copy(vals_flat.at[pl.ds(r * 128, 128)],
                             idx_sh.at[slots_v.at[r]], sem_s).wait()
            return carry

        lax.fori_loop(0, 8, _scat, jnp.int32(0))

    plsc.subcore_barrier()

    @pl.when(jnp.logical_and(cid == 0, sid == 0))
    def _emit():
        pltpu.sync_copy(idx_sh.at[pl.ds(0, K)], idx_hbm)


_sc_idx = pl.kernel(
    _idx_body,
    out_type=jax.ShapeDtypeStruct((K,), jnp.int32),
    mesh=plsc.VectorSubcoreMesh(core_axis_name="c", subcore_axis_name="s"),
    compiler_params=pltpu.CompilerParams(needs_layout_passes=False),
    scratch_types=[
        pltpu.VMEM((CHUNK,), jnp.int32),
        pltpu.VMEM((L,), jnp.int32),
        pltpu.VMEM((CHUNK,), jnp.int32),
        pltpu.VMEM((8, 128), jnp.int32),
        pltpu.VMEM((NS, L), jnp.int32),
        pltpu.VMEM_SHARED((NS, L), jnp.int32),
        pltpu.VMEM_SHARED((K + L,), jnp.int32),
        pltpu.SemaphoreType.DMA,
    ],
)


NBUF = 4


def _tc_body(idx_sm, out_hbm, loss_ref, sel_ref,
             buf0, buf1, buf2, buf3, sem0, sem1, sem2, sem3):
    i = pl.program_id(0)
    bufs = (buf0, buf1, buf2, buf3)
    sems = (sem0, sem1, sem2, sem3)

    def _issue(block, buf, sem):
        for r in range(BR):
            pltpu.make_async_copy(
                out_hbm.at[pl.ds(idx_sm[block * BR + r], 1), :],
                buf.at[pl.ds(r, 1), :], sem).start()

    def _drain(block, buf, sem):
        for r in range(BR):
            pltpu.make_async_copy(
                out_hbm.at[pl.ds(idx_sm[block * BR + r], 1), :],
                buf.at[pl.ds(r, 1), :], sem).wait()

    @pl.when(i == 0)
    def _prologue():
        loss_ref[0, 0] = jnp.float32(0.0)
        for b in range(NBUF - 1):
            _issue(b, bufs[b], sems[b])

    for p in range(NBUF):
        @pl.when(jnp.logical_and(i + NBUF - 1 < NBLK,
                                 (i + NBUF - 1) % NBUF == p))
        def _issue_ahead(p=p):
            _issue(i + NBUF - 1, bufs[p], sems[p])

        @pl.when(i % NBUF == p)
        def _consume(p=p):
            _drain(i, bufs[p], sems[p])
            sel_ref[...] = bufs[p][...]
            loss_ref[0, 0] += jnp.sum(bufs[p][...])


_tc_gather = pl.pallas_call(
    _tc_body,
    grid_spec=pltpu.PrefetchScalarGridSpec(
        num_scalar_prefetch=1,
        grid=(NBLK,),
        in_specs=[pl.BlockSpec(memory_space=pltpu.MemorySpace.HBM)],
        out_specs=[
            pl.BlockSpec((1, 1), lambda i, idx: (0, 0),
                         memory_space=pltpu.MemorySpace.SMEM),
            pl.BlockSpec((BR, D), lambda i, idx: (i, 0)),
        ],
        scratch_shapes=[
            pltpu.VMEM((BR, D), jnp.float32),
            pltpu.VMEM((BR, D), jnp.float32),
            pltpu.VMEM((BR, D), jnp.float32),
            pltpu.VMEM((BR, D), jnp.float32),
            pltpu.SemaphoreType.DMA,
            pltpu.SemaphoreType.DMA,
            pltpu.SemaphoreType.DMA,
            pltpu.SemaphoreType.DMA,
        ],
    ),
    out_shape=(
        jax.ShapeDtypeStruct((1, 1), jnp.float32),
        jax.ShapeDtypeStruct((K, D), jnp.float32),
    ),
)


def kernel(outputs, labels, num_masks):
    nm_arr = jnp.full((L,), num_masks, dtype=jnp.int32)
    idx = _sc_idx(labels, nm_arr)
    loss_sum, sel = _tc_gather(idx, outputs)
    loss = loss_sum[0, 0] * jnp.float32(1.0 / (K * D))
    return loss, sel


# speedup vs baseline: 1.3682x; 1.2029x over previous
import functools

import jax
import jax.numpy as jnp
from jax import lax
from jax.experimental import pallas as pl
from jax.experimental.pallas import tpu as pltpu
from jax.experimental.pallas import tpu_sc as plsc

N = 16384
D = 4096
K = 512
NC = 2
NS = 16
L = 16
CHUNK = N // NS
RPT = K // (NC * NS)
QR = RPT // 4

_ONE = lambda: jnp.full((L,), 1, jnp.int32)
_ZERO = lambda: jnp.full((L,), 0, jnp.int32)


def _sc_body(outputs_hbm, labels_hbm, nm_hbm, loss_hbm, sel_hbm,
             lab_v, nm_v, vals_flat, slots_v, idx_v, rows_v, tmp_v,
             cnts_all_v, msum, counts_sh, idx_sh,
             sem_l, sem_c, semg0, semg1, semg2, semg3, semw, sem_s):
    cid = lax.axis_index("c")
    sid = lax.axis_index("s")
    r0 = cid * (NS * RPT) + sid * RPT

    ld_lab = pltpu.async_copy(labels_hbm.at[pl.ds(sid * CHUNK, CHUNK)], lab_v,
                              sem_l)
    ld_nm = pltpu.async_copy(nm_hbm, nm_v, sem_l)
    semg = (semg0, semg1, semg2, semg3)
    spec = [pltpu.async_copy(outputs_hbm.at[pl.ds(N - K + r0 + k * QR, QR)],
                             rows_v.at[pl.ds(k * QR, QR)], semg[k])
            for k in range(4)]

    msum[0] = jnp.int32(0)
    plsc.subcore_barrier()

    ld_lab.wait()
    ld_nm.wait()

    def _count(r, acc):
        for t in range(8):
            v = lab_v[pl.ds((r * 8 + t) * L, L)]
            acc = acc + jnp.where(v != -100, _ONE(), _ZERO())
        return acc

    acc0 = lax.fori_loop(0, 8, _count, jnp.zeros((L,), jnp.int32))
    count = jnp.sum(acc0)
    nm = nm_v[...][0]
    nm_eff = jnp.minimum(jnp.maximum(nm, K), N)

    nm_v[...] = jnp.full((L,), count, jnp.int32)
    wcnt = pltpu.async_copy(nm_v, counts_sh.at[sid], sem_c)
    for s in range(NS):
        plsc.fetch_and_add(msum.at[0], count, subcore_id=s)

    def _sum_quarter(k, acc):
        def _sum(cb, acc2):
            base = cb * 4 * L
            for r in range(QR):
                for c in range(4):
                    acc2 = acc2 + rows_v[k * QR + r, pl.ds(base + c * L, L)]
            return acc2

        return lax.fori_loop(0, D // (4 * L), _sum, acc)

    accf = jnp.zeros((L,), jnp.float32)
    writes = []
    for k in range(4):
        spec[k].wait()
        writes.append(
            pltpu.async_copy(rows_v.at[pl.ds(k * QR, QR)],
                             sel_hbm.at[pl.ds(r0 + k * QR, QR)], semw))
        accf = _sum_quarter(k, accf)
    tmp_v[...] = accf
    wcnt.wait()
    for w in writes:
        w.wait()
    plsc.subcore_barrier()

    m = msum[0]
    fastg = jnp.logical_and(m == N, nm_eff == K)

    @pl.when(jnp.logical_not(fastg))
    def _slow_path():
        pltpu.sync_copy(counts_sh, cnts_all_v)
        cvals = [cnts_all_v[j][0] for j in range(NS)]
        zero = jnp.int32(0)
        oa = functools.reduce(
            lambda a, b: a + b,
            [jnp.where(jnp.int32(j) > sid, cvals[j], zero) for j in range(NS)])
        cs = functools.reduce(
            lambda a, b: a + b,
            [jnp.where(jnp.int32(j) == sid, cvals[j], zero)
             for j in range(NS)])
        ob = m - oa - cs
        za = (N - (sid + 1) * CHUNK) - oa

        def _stage(j, carry):
            vals_flat[pl.ds(j * L, L)] = (
                sid * CHUNK + j * L + lax.iota(jnp.int32, L))
            return carry

        lax.fori_loop(0, CHUNK // L, _stage, jnp.int32(0))

        fast = jnp.logical_and(
            jnp.logical_and(oa == 0, za == 0),
            jnp.logical_and(cs == CHUNK, nm_eff == K))

        @pl.when(fast)
        def _linear_indices():
            pltpu.sync_copy(vals_flat.at[pl.ds(CHUNK - K, K)],
                            idx_sh.at[pl.ds(0, K)])

        @pl.when(jnp.logical_and(jnp.minimum(oa, m + za) < nm_eff,
                                 jnp.logical_not(fast)))
        def _scatter_indices():
            def _fill(j, carry):
                v = lab_v[pl.ds(j * L, L)]
                is1 = v != -100
                ones = jnp.where(is1, _ONE(), _ZERO())
                up_incl = carry + plsc.cumsum(ones)
                ones_after = m - up_incl
                ivec = vals_flat[pl.ds(j * L, L)]
                zeros_after = (N - 1 - ivec) - ones_after
                rank = jnp.where(is1, ones_after, m + zeros_after)
                slot = (nm_eff - 1) - rank
                dump = K + lax.iota(jnp.int32, L)
                scat = jnp.where(slot >= 0,
                                 jnp.where(slot < K, slot, dump), dump)
                slots_v[j // 8, pl.ds((j % 8) * L, L)] = scat
                return carry + jnp.sum(ones)

            lax.fori_loop(0, CHUNK // L, _fill, ob)

            def _scat(r, carry):
                pltpu.async_copy(vals_flat.at[pl.ds(r * 128, 128)],
                                 idx_sh.at[slots_v.at[r]], sem_s).wait()
                return carry

            lax.fori_loop(0, 8, _scat, jnp.int32(0))

        plsc.subcore_barrier()

        pltpu.sync_copy(idx_sh.at[pl.ds(r0, RPT)], idx_v)
        g0 = pltpu.async_copy(outputs_hbm.at[idx_v.at[pl.ds(0, RPT // 2)]],
                              rows_v.at[pl.ds(0, RPT // 2)], semg0)
        g1 = pltpu.async_copy(
            outputs_hbm.at[idx_v.at[pl.ds(RPT // 2, RPT // 2)]],
            rows_v.at[pl.ds(RPT // 2, RPT // 2)], semg1)
        g0.wait()
        g1.wait()
        acc2 = jnp.zeros((L,), jnp.float32)
        rewrites = []
        for k in range(4):
            rewrites.append(
                pltpu.async_copy(rows_v.at[pl.ds(k * QR, QR)],
                                 sel_hbm.at[pl.ds(r0 + k * QR, QR)], semw))
            acc2 = _sum_quarter(k, acc2)
        for w in rewrites:
            w.wait()
        tmp_v[...] = acc2

    pltpu.sync_copy(tmp_v, loss_hbm.at[cid, sid])


_sc_call = pl.kernel(
    _sc_body,
    out_type=(
        jax.ShapeDtypeStruct((NC, NS, L), jnp.float32),
        jax.ShapeDtypeStruct((K, D), jnp.float32),
    ),
    mesh=plsc.VectorSubcoreMesh(core_axis_name="c", subcore_axis_name="s"),
    compiler_params=pltpu.CompilerParams(needs_layout_passes=False),
    scratch_types=[
        pltpu.VMEM((CHUNK,), jnp.int32),
        pltpu.VMEM((L,), jnp.int32),
        pltpu.VMEM((CHUNK,), jnp.int32),
        pltpu.VMEM((8, 128), jnp.int32),
        pltpu.VMEM((RPT,), jnp.int32),
        pltpu.VMEM((RPT, D), jnp.float32),
        pltpu.VMEM((L,), jnp.float32),
        pltpu.VMEM((NS, L), jnp.int32),
        pltpu.SMEM((8,), jnp.int32),
        pltpu.VMEM_SHARED((NS, L), jnp.int32),
        pltpu.VMEM_SHARED((K + L,), jnp.int32),
        pltpu.SemaphoreType.DMA,
        pltpu.SemaphoreType.DMA,
        pltpu.SemaphoreType.DMA,
        pltpu.SemaphoreType.DMA,
        pltpu.SemaphoreType.DMA,
        pltpu.SemaphoreType.DMA,
        pltpu.SemaphoreType.DMA,
        pltpu.SemaphoreType.DMA,
    ],
)


def kernel(outputs, labels, num_masks):
    nm_arr = jnp.full((L,), num_masks, dtype=jnp.int32)
    loss_parts, sel = _sc_call(outputs, labels, nm_arr)
    loss = jnp.sum(loss_parts) * jnp.float32(1.0 / (K * D))
    return loss, sel
